# baseline (speedup 1.0000x reference)
import functools

import jax
import jax.numpy as jnp
from jax import lax
from jax.experimental import pallas as pl
from jax.experimental.pallas import tpu as pltpu
from jax.experimental.pallas import tpu_sc as plsc

N = 1048576
NC = 2
NS = 16
NW = NC * NS
C = N // NW
L = 16
J = C // L
ND = 4
SZ = C // ND
SZV = SZ // L

_mesh = plsc.VectorSubcoreMesh(core_axis_name="c", subcore_axis_name="s",
                               num_cores=NC)


@functools.partial(
    pl.kernel,
    mesh=_mesh,
    out_type=jax.ShapeDtypeStruct((NW, L), jnp.int32),
    scratch_types=[
        pltpu.VMEM((C + L,), jnp.float32),
        pltpu.VMEM((L,), jnp.int32),
        pltpu.SemaphoreType.DMA,
        pltpu.SemaphoreType.DMA,
        pltpu.SemaphoreType.DMA,
        pltpu.SemaphoreType.DMA,
    ],
)
def _uc_kernel(x_hbm, flag_hbm, buf, stage_f, sem0, sem1, sem2, sem3):
    c = lax.axis_index("c")
    s = lax.axis_index("s")
    w = c * NS + s
    base = w * C
    ones = jnp.full((L,), 1, jnp.int32)
    zeros = jnp.full((L,), 0, jnp.int32)
    sems = [sem0, sem1, sem2, sem3]

    for d in range(ND - 1):
        pltpu.async_copy(x_hbm.at[pl.ds(base + d * SZ, SZ + 8)],
                         buf.at[pl.ds(d * SZ, SZ + 8)], sems[d])
    _last = (ND - 1) * SZ

    @pl.when(w < NW - 1)
    def _():
        pltpu.async_copy(x_hbm.at[pl.ds(base + _last, SZ + 8)],
                         buf.at[pl.ds(_last, SZ + 8)], sems[ND - 1])

    @pl.when(w == NW - 1)
    def _():
        pltpu.async_copy(x_hbm.at[pl.ds(base + _last, SZ)],
                         buf.at[pl.ds(_last, SZ)], sems[ND - 1])

    U = 16

    def compute_sub(d, carry):
        def body(j, carry):
            acc, cnt = carry
            for k in range(U):
                off = d * SZ + (j * U + k) * L
                a = buf[pl.ds(off, L)]
                b = buf[pl.ds(off + 1, L)]
                neq = a != b
                acc = acc & (b == b)
                cnt = cnt + jnp.where(neq, ones, zeros)
            return acc, cnt

        return lax.fori_loop(0, SZV // U, body, carry)

    pltpu.make_async_copy(x_hbm.at[pl.ds(base, SZ + 8)],
                          buf.at[pl.ds(0, SZ + 8)], sems[0]).wait()
    v0 = buf[pl.ds(0, L)]
    carry = (v0 == v0, jnp.zeros((L,), jnp.int32))
    carry = compute_sub(0, carry)
    for d in range(1, ND - 1):
        pltpu.make_async_copy(x_hbm.at[pl.ds(base + d * SZ, SZ + 8)],
                              buf.at[pl.ds(d * SZ, SZ + 8)], sems[d]).wait()
        carry = compute_sub(d, carry)

    @pl.when(w < NW - 1)
    def _():
        pltpu.make_async_copy(x_hbm.at[pl.ds(base + _last, SZ + 8)],
                              buf.at[pl.ds(_last, SZ + 8)], sems[ND - 1]).wait()

    @pl.when(w == NW - 1)
    def _():
        pltpu.make_async_copy(x_hbm.at[pl.ds(base + _last, SZ)],
                              buf.at[pl.ds(_last, SZ)], sems[ND - 1]).wait()
        buf[pl.ds(C, L)] = buf[pl.ds(C - 1, L)]

    acc, cnt = compute_sub(ND - 1, carry)

    f = jnp.minimum(jnp.where(acc, ones, zeros),
                    jnp.where(cnt == cnt, ones, zeros))
    stage_f[...] = f
    pltpu.sync_copy(stage_f, flag_hbm.at[w])


def kernel(x):
    flags = _uc_kernel(x)
    return jnp.all(flags != 0)


# speedup vs baseline: 1.0099x; 1.0099x over previous
import functools

import jax
import jax.numpy as jnp
from jax import lax
from jax.experimental import pallas as pl
from jax.experimental.pallas import tpu as pltpu
from jax.experimental.pallas import tpu_sc as plsc

N = 1048576
NC = 2
NS = 16
NW = NC * NS
C = N // NW
L = 16
J = C // L
ND = 4
SZ = C // ND
SZV = SZ // L

_mesh = plsc.VectorSubcoreMesh(core_axis_name="c", subcore_axis_name="s",
                               num_cores=NC)


@functools.partial(
    pl.kernel,
    mesh=_mesh,
    out_type=jax.ShapeDtypeStruct((NW, L), jnp.int32),
    scratch_types=[
        pltpu.VMEM((C + L,), jnp.float32),
        pltpu.VMEM((L,), jnp.int32),
        pltpu.SemaphoreType.DMA,
        pltpu.SemaphoreType.DMA,
        pltpu.SemaphoreType.DMA,
        pltpu.SemaphoreType.DMA,
    ],
)
def _uc_kernel(x_hbm, flag_hbm, buf, stage_f, sem0, sem1, sem2, sem3):
    c = lax.axis_index("c")
    s = lax.axis_index("s")
    w = c * NS + s
    base = w * C
    ones = jnp.full((L,), 1, jnp.int32)
    zeros = jnp.full((L,), 0, jnp.int32)
    sems = [sem0, sem1, sem2, sem3]

    for d in range(ND - 1):
        pltpu.async_copy(x_hbm.at[pl.ds(base + d * SZ, SZ + 8)],
                         buf.at[pl.ds(d * SZ, SZ + 8)], sems[d])
    _last = (ND - 1) * SZ

    @pl.when(w < NW - 1)
    def _():
        pltpu.async_copy(x_hbm.at[pl.ds(base + _last, SZ + 8)],
                         buf.at[pl.ds(_last, SZ + 8)], sems[ND - 1])

    @pl.when(w == NW - 1)
    def _():
        pltpu.async_copy(x_hbm.at[pl.ds(base + _last, SZ)],
                         buf.at[pl.ds(_last, SZ)], sems[ND - 1])

    U = 8

    def compute_sub(d, carry0):
        @plsc.parallel_loop(0, SZV, step=U, unroll=2, carry=carry0)
        def body(v, carry):
            acc, cnt = carry
            for k in range(U):
                off = d * SZ + (v + k) * L
                a = buf[pl.ds(off, L)]
                b = buf[pl.ds(off + 1, L)]
                neq = a != b
                acc = acc & (b == b)
                cnt = cnt + jnp.where(neq, ones, zeros)
            return acc, cnt

        return body

    pltpu.make_async_copy(x_hbm.at[pl.ds(base, SZ + 8)],
                          buf.at[pl.ds(0, SZ + 8)], sems[0]).wait()
    v0 = buf[pl.ds(0, L)]
    carry = (v0 == v0, jnp.zeros((L,), jnp.int32))
    carry = compute_sub(0, carry)
    for d in range(1, ND - 1):
        pltpu.make_async_copy(x_hbm.at[pl.ds(base + d * SZ, SZ + 8)],
                              buf.at[pl.ds(d * SZ, SZ + 8)], sems[d]).wait()
        carry = compute_sub(d, carry)

    @pl.when(w < NW - 1)
    def _():
        pltpu.make_async_copy(x_hbm.at[pl.ds(base + _last, SZ + 8)],
                              buf.at[pl.ds(_last, SZ + 8)], sems[ND - 1]).wait()

    @pl.when(w == NW - 1)
    def _():
        pltpu.make_async_copy(x_hbm.at[pl.ds(base + _last, SZ)],
                              buf.at[pl.ds(_last, SZ)], sems[ND - 1]).wait()
        buf[pl.ds(C, L)] = buf[pl.ds(C - 1, L)]

    acc, cnt = compute_sub(ND - 1, carry)

    f = jnp.minimum(jnp.where(acc, ones, zeros),
                    jnp.where(cnt == cnt, ones, zeros))
    stage_f[...] = f
    pltpu.sync_copy(stage_f, flag_hbm.at[w])


def kernel(x):
    flags = _uc_kernel(x)
    return jnp.all(flags != 0)
